# Initial kernel scaffold; baseline (speedup 1.0000x reference)
import jax
import jax.numpy as jnp
from jax.experimental import pallas as pl


def kernel(x, Wb, bb, Wg, bg, Wn, bn, W1, b1, W2, b2):
    raise NotImplementedError("write your pallas kernel here")


# speedup vs baseline: 1.7283x; 1.7283x over previous
import functools

import jax
import jax.numpy as jnp
from jax.experimental import pallas as pl
from jax.experimental.pallas import tpu as pltpu

E = 16
K = 2
TEMP = 1.5
MIN_NOISE = 0.01

B = 2048
D = 768
H = 3072
N = B * K
TM = 256
NT = N // TM
NW = NT + E - 1
TB = 256

NEG = -3.0e38


def _gate_body(x_ref, wb_ref, bb_ref, wg_ref, bg_ref, wn_ref, bn_ref,
               noise_ref, feats_ref, idx_ref, wts_ref):
    x = x_ref[...]
    feats = jnp.dot(x, wb_ref[...], preferred_element_type=jnp.float32)
    feats = feats + bb_ref[...]
    feats_ref[...] = feats
    raw = jnp.dot(feats, wg_ref[...], preferred_element_type=jnp.float32)
    raw = raw + bg_ref[...]
    nl = jnp.dot(feats, wn_ref[...], preferred_element_type=jnp.float32)
    nl = nl + bn_ref[...]
    sigma = jax.nn.softplus(nl) + MIN_NOISE
    scores = raw + sigma * noise_ref[...]
    iota = jax.lax.broadcasted_iota(jnp.int32, scores.shape, 1)
    m1 = jnp.max(scores, axis=1, keepdims=True)
    i1 = jnp.min(jnp.where(scores == m1, iota, E), axis=1, keepdims=True)
    s2 = jnp.where(iota == i1, NEG, scores)
    m2 = jnp.max(s2, axis=1, keepdims=True)
    i2 = jnp.min(jnp.where(s2 == m2, iota, E), axis=1, keepdims=True)
    e2 = jnp.exp((m2 - m1) / TEMP)
    w1 = 1.0 / (1.0 + e2)
    w2 = 1.0 - w1
    idx_ref[...] = jnp.concatenate([i1, i2], axis=1)
    wts_ref[...] = jnp.concatenate([w1, w2], axis=1)


def _gate(x, Wb, bb, Wg, bg, Wn, bn, noise):
    grid = (B // TB,)
    return pl.pallas_call(
        _gate_body,
        grid=grid,
        in_specs=[
            pl.BlockSpec((TB, D), lambda i: (i, 0)),
            pl.BlockSpec((D, D), lambda i: (0, 0)),
            pl.BlockSpec((1, D), lambda i: (0, 0)),
            pl.BlockSpec((D, E), lambda i: (0, 0)),
            pl.BlockSpec((1, E), lambda i: (0, 0)),
            pl.BlockSpec((D, E), lambda i: (0, 0)),
            pl.BlockSpec((1, E), lambda i: (0, 0)),
            pl.BlockSpec((TB, E), lambda i: (i, 0)),
        ],
        out_specs=[
            pl.BlockSpec((TB, D), lambda i: (i, 0)),
            pl.BlockSpec((TB, K), lambda i: (i, 0)),
            pl.BlockSpec((TB, K), lambda i: (i, 0)),
        ],
        out_shape=[
            jax.ShapeDtypeStruct((B, D), jnp.float32),
            jax.ShapeDtypeStruct((B, K), jnp.int32),
            jax.ShapeDtypeStruct((B, K), jnp.float32),
        ],
    )(x, Wb, bb.reshape(1, D), Wg, bg.reshape(1, E), Wn, bn.reshape(1, E),
      noise)


def _ffn_body(tile_ref, exp_ref, lo_ref, hi_ref, first_ref,
              xs_ref, w1_ref, b1_ref, w2_ref, b2_ref, ys_ref):
    u = pl.program_id(0)
    lo = lo_ref[u]
    hi = hi_ref[u]
    first = first_ref[u]
    xs = xs_ref[...]
    h = jnp.dot(xs, w1_ref[0], preferred_element_type=jnp.float32)
    h = jnp.maximum(h + b1_ref[0], 0.0)
    y = jnp.dot(h, w2_ref[0], preferred_element_type=jnp.float32)
    y = y + b2_ref[0]
    rows = jax.lax.broadcasted_iota(jnp.int32, (TM, D), 0)
    mask = (rows >= lo) & (rows < hi)
    prev = jnp.where(first == 1, 0.0, ys_ref[...])
    ys_ref[...] = jnp.where(mask, y, prev)


def _ffn(xs, W1, b1, W2, b2, tile_id, exp_id, lo, hi, first):
    grid_spec = pltpu.PrefetchScalarGridSpec(
        num_scalar_prefetch=5,
        grid=(NW,),
        in_specs=[
            pl.BlockSpec((TM, D), lambda u, t, e, l, h, f: (t[u], 0)),
            pl.BlockSpec((1, D, H), lambda u, t, e, l, h, f: (e[u], 0, 0)),
            pl.BlockSpec((1, 1, H), lambda u, t, e, l, h, f: (e[u], 0, 0)),
            pl.BlockSpec((1, H, D), lambda u, t, e, l, h, f: (e[u], 0, 0)),
            pl.BlockSpec((1, 1, D), lambda u, t, e, l, h, f: (e[u], 0, 0)),
        ],
        out_specs=pl.BlockSpec((TM, D), lambda u, t, e, l, h, f: (t[u], 0)),
    )
    return pl.pallas_call(
        _ffn_body,
        grid_spec=grid_spec,
        out_shape=jax.ShapeDtypeStruct((N, D), jnp.float32),
    )(tile_id, exp_id, lo, hi, first, xs, W1,
      b1.reshape(E, 1, H), W2, b2.reshape(E, 1, D))


def _route(idx, wts):
    flat_e = idx.reshape(-1)
    flat_tok = jnp.arange(N, dtype=jnp.int32) // K
    perm = jnp.argsort(flat_e, stable=True)
    tok_sorted = flat_tok[perm]
    counts = jnp.sum(flat_e[:, None] == jnp.arange(E)[None, :], axis=0)
    bounds = jnp.concatenate(
        [jnp.zeros((1,), jnp.int32), jnp.cumsum(counts).astype(jnp.int32)])
    pos = jnp.zeros((N,), jnp.int32).at[perm].set(
        jnp.arange(N, dtype=jnp.int32)).reshape(B, K)
    return tok_sorted, bounds, pos


def _work_units(bounds):
    cnt = bounds[1:] - bounds[:-1]
    t_lo = bounds[:-1] // TM
    t_hi = (bounds[1:] + TM - 1) // TM
    n_units = jnp.where(cnt > 0, t_hi - t_lo, 0)
    slot_end = jnp.cumsum(n_units)
    slot_start = slot_end - n_units
    total = slot_end[-1]
    u = jnp.arange(NW, dtype=jnp.int32)
    e_u = jnp.searchsorted(slot_end, u, side='right').astype(jnp.int32)
    valid = u < total
    e_u = jnp.clip(e_u, 0, E - 1)
    tile = t_lo[e_u] + (u - slot_start[e_u])
    tile = jnp.where(valid, tile, NT - 1).astype(jnp.int32)
    lo = jnp.clip(bounds[e_u] - tile * TM, 0, TM)
    hi = jnp.clip(bounds[e_u + 1] - tile * TM, 0, TM)
    lo = jnp.where(valid, lo, 0).astype(jnp.int32)
    hi = jnp.where(valid, hi, 0).astype(jnp.int32)
    prev_tile = jnp.concatenate([jnp.full((1,), -1, jnp.int32), tile[:-1]])
    first = (valid & (tile != prev_tile)).astype(jnp.int32)
    return tile, e_u.astype(jnp.int32), lo, hi, first


def kernel(x, Wb, bb, Wg, bg, Wn, bn, W1, b1, W2, b2):
    noise = jax.random.normal(jax.random.key(42), (B, E), dtype=jnp.float32)
    feats, idx, wts = _gate(x, Wb, bb, Wg, bg, Wn, bn, noise)
    tok_sorted, bounds, pos = _route(idx, wts)
    tile_id, exp_id, lo, hi, first = _work_units(bounds)
    xs = feats[tok_sorted]
    ys = _ffn(xs, W1, b1, W2, b2, tile_id, exp_id, lo, hi, first)
    out = wts[:, 0:1] * ys[pos[:, 0]] + wts[:, 1:2] * ys[pos[:, 1]]
    return out


# speedup vs baseline: 1.8192x; 1.0526x over previous
import functools

import jax
import jax.numpy as jnp
from jax import lax
from jax.experimental import pallas as pl
from jax.experimental.pallas import tpu as pltpu
from jax.experimental.pallas import tpu_sc as plsc

E = 16
K = 2
TEMP = 1.5
MIN_NOISE = 0.01

B = 2048
D = 768
H = 3072
N = B * K
TM = 256
NT = N // TM
NW = NT + E - 1
TB = 256

NEG = -3.0e38


def _gate_body(x_ref, wb_ref, bb_ref, wg_ref, bg_ref, wn_ref, bn_ref,
               noise_ref, feats_ref, idx_ref, wts_ref):
    x = x_ref[...]
    feats = jnp.dot(x, wb_ref[...], preferred_element_type=jnp.float32)
    feats = feats + bb_ref[...]
    feats_ref[...] = feats
    raw = jnp.dot(feats, wg_ref[...], preferred_element_type=jnp.float32)
    raw = raw + bg_ref[...]
    nl = jnp.dot(feats, wn_ref[...], preferred_element_type=jnp.float32)
    nl = nl + bn_ref[...]
    sigma = jax.nn.softplus(nl) + MIN_NOISE
    scores = raw + sigma * noise_ref[...]
    iota = jax.lax.broadcasted_iota(jnp.int32, scores.shape, 1)
    m1 = jnp.max(scores, axis=1, keepdims=True)
    i1 = jnp.min(jnp.where(scores == m1, iota, E), axis=1, keepdims=True)
    s2 = jnp.where(iota == i1, NEG, scores)
    m2 = jnp.max(s2, axis=1, keepdims=True)
    i2 = jnp.min(jnp.where(s2 == m2, iota, E), axis=1, keepdims=True)
    e2 = jnp.exp((m2 - m1) / TEMP)
    w1 = 1.0 / (1.0 + e2)
    w2 = 1.0 - w1
    idx_ref[...] = jnp.concatenate([i1, i2], axis=1)
    wts_ref[...] = jnp.concatenate([w1, w2], axis=1)


def _gate(x, Wb, bb, Wg, bg, Wn, bn, noise):
    grid = (B // TB,)
    return pl.pallas_call(
        _gate_body,
        grid=grid,
        in_specs=[
            pl.BlockSpec((TB, D), lambda i: (i, 0)),
            pl.BlockSpec((D, D), lambda i: (0, 0)),
            pl.BlockSpec((1, D), lambda i: (0, 0)),
            pl.BlockSpec((D, E), lambda i: (0, 0)),
            pl.BlockSpec((1, E), lambda i: (0, 0)),
            pl.BlockSpec((D, E), lambda i: (0, 0)),
            pl.BlockSpec((1, E), lambda i: (0, 0)),
            pl.BlockSpec((TB, E), lambda i: (i, 0)),
        ],
        out_specs=[
            pl.BlockSpec((TB, D), lambda i: (i, 0)),
            pl.BlockSpec((TB, K), lambda i: (i, 0)),
            pl.BlockSpec((TB, K), lambda i: (i, 0)),
        ],
        out_shape=[
            jax.ShapeDtypeStruct((B, D), jnp.float32),
            jax.ShapeDtypeStruct((B, K), jnp.int32),
            jax.ShapeDtypeStruct((B, K), jnp.float32),
        ],
    )(x, Wb, bb.reshape(1, D), Wg, bg.reshape(1, E), Wn, bn.reshape(1, E),
      noise)


def _ffn_body(tile_ref, exp_ref, lo_ref, hi_ref, first_ref,
              xs_ref, w1_ref, b1_ref, w2_ref, b2_ref, ys_ref):
    u = pl.program_id(0)
    lo = lo_ref[u]
    hi = hi_ref[u]
    first = first_ref[u]
    xs = xs_ref[...]
    h = jnp.dot(xs, w1_ref[0], preferred_element_type=jnp.float32)
    h = jnp.maximum(h + b1_ref[0], 0.0)
    y = jnp.dot(h, w2_ref[0], preferred_element_type=jnp.float32)
    y = y + b2_ref[0]
    rows = jax.lax.broadcasted_iota(jnp.int32, (TM, D), 0)
    mask = (rows >= lo) & (rows < hi)
    prev = jnp.where(first == 1, 0.0, ys_ref[...])
    ys_ref[...] = jnp.where(mask, y, prev)


def _ffn(xs, W1, b1, W2, b2, tile_id, exp_id, lo, hi, first):
    grid_spec = pltpu.PrefetchScalarGridSpec(
        num_scalar_prefetch=5,
        grid=(NW,),
        in_specs=[
            pl.BlockSpec((TM, D), lambda u, t, e, l, h, f: (t[u], 0)),
            pl.BlockSpec((1, D, H), lambda u, t, e, l, h, f: (e[u], 0, 0)),
            pl.BlockSpec((1, 1, H), lambda u, t, e, l, h, f: (e[u], 0, 0)),
            pl.BlockSpec((1, H, D), lambda u, t, e, l, h, f: (e[u], 0, 0)),
            pl.BlockSpec((1, 1, D), lambda u, t, e, l, h, f: (e[u], 0, 0)),
        ],
        out_specs=pl.BlockSpec((TM, D), lambda u, t, e, l, h, f: (t[u], 0)),
    )
    return pl.pallas_call(
        _ffn_body,
        grid_spec=grid_spec,
        out_shape=jax.ShapeDtypeStruct((N, D), jnp.float32),
    )(tile_id, exp_id, lo, hi, first, xs, W1,
      b1.reshape(E, 1, H), W2, b2.reshape(E, 1, D))


NC = 2
NS = 16
CH = N // NS
CHR = CH // 128


def _route_sc_body(idx_hbm, tok_hbm, toks_hbm, pos_hbm, cnt_hbm,
                   chunk_v, dst_v, rank_v, hist_v, cnt_v,
                   tmp_v, tmp1d_v, sh_hist, sh_tok):
    c = lax.axis_index("c")
    s = lax.axis_index("s")
    base = s * CH

    @pl.when(c == 0)
    def _():
        iota = lax.iota(jnp.int32, NS)
        zc = jnp.zeros((16,), jnp.int32)
        ones = zc + 1

        def eq01(a, b):
            d = a - b
            return ones - jnp.minimum(d * d, ones)

        pltpu.sync_copy(idx_hbm.at[pl.ds(s * CHR, CHR)], chunk_v)
        hist = zc
        for k in range(CHR):
            for g in range(8):
                ev = chunk_v[k, pl.ds(g * 16, 16)]
                rank = zc
                for j in range(16):
                    bj = jnp.broadcast_to(ev[j], (16,))
                    gtj = jnp.minimum(jnp.maximum(iota - j, zc), ones)
                    hist = hist + eq01(iota, bj)
                    rank = rank + eq01(ev, bj) * gtj
                rank_v[k, pl.ds(g * 16, 16)] = rank
        hist_v[...] = hist
        pltpu.sync_copy(hist_v, sh_hist.at[s])
        plsc.subcore_barrier()
        sv = jnp.broadcast_to(s, (16,))
        basev = zc
        totv = zc
        for r in range(NS):
            pltpu.sync_copy(sh_hist.at[r], hist_v)
            row = hist_v[...]
            mine = jnp.minimum(jnp.maximum(sv - r, zc), ones)
            basev = basev + row * mine
            totv = totv + row
        excl = zc
        acc = totv[0] * 0
        for e in range(E):
            excl = excl + eq01(iota, zc + e) * jnp.broadcast_to(acc, (16,))
            acc = acc + totv[e]
        startv = excl + basev

        @pl.when(s == 0)
        def _():
            cnt_v[...] = totv
            pltpu.sync_copy(cnt_v, cnt_hbm)

        for k in range(CHR):
            for g in range(8):
                ev = chunk_v[k, pl.ds(g * 16, 16)]
                rank = rank_v[k, pl.ds(g * 16, 16)]
                startlane = zc
                histg = zc
                for e in range(E):
                    me = eq01(ev, zc + e)
                    startlane = startlane + me * jnp.broadcast_to(
                        startv[e], (16,))
                for j in range(16):
                    bj = jnp.broadcast_to(ev[j], (16,))
                    histg = histg + eq01(iota, bj)
                startv = startv + histg
                dst_v[k, pl.ds(g * 16, 16)] = startlane + rank
        pltpu.sync_copy(tok_hbm.at[pl.ds(s * CHR, CHR)], tmp_v)
        for k in range(CHR):
            pltpu.sync_copy(dst_v.at[k], pos_hbm.at[pl.ds(base + k * 128, 128)])
            pltpu.sync_copy(tmp_v.at[k], sh_tok.at[dst_v.at[k]])
        plsc.subcore_barrier()
        pltpu.sync_copy(sh_tok.at[pl.ds(base, CH)], tmp1d_v)
        pltpu.sync_copy(tmp1d_v, toks_hbm.at[pl.ds(base, CH)])


def _route_sc(idx_flat, tok_flat):
    mesh = plsc.VectorSubcoreMesh(core_axis_name="c", subcore_axis_name="s",
                                  num_cores=NC, num_subcores=NS)
    f = pl.kernel(
        _route_sc_body,
        out_type=[
            jax.ShapeDtypeStruct((N,), jnp.int32),
            jax.ShapeDtypeStruct((N,), jnp.int32),
            jax.ShapeDtypeStruct((E,), jnp.int32),
        ],
        mesh=mesh,
        scratch_types=[
            pltpu.VMEM((CHR, 128), jnp.int32),
            pltpu.VMEM((CHR, 128), jnp.int32),
            pltpu.VMEM((CHR, 128), jnp.int32),
            pltpu.VMEM((NS,), jnp.int32),
            pltpu.VMEM((E,), jnp.int32),
            pltpu.VMEM((CHR, 128), jnp.int32),
            pltpu.VMEM((CH,), jnp.int32),
            pltpu.VMEM_SHARED((NS, NS), jnp.int32),
            pltpu.VMEM_SHARED((N,), jnp.int32),
        ],
    )
    return f(idx_flat.reshape(N // 128, 128), tok_flat.reshape(N // 128, 128))


def _route(idx, wts):
    flat_e = idx.reshape(-1)
    flat_tok = jnp.arange(N, dtype=jnp.int32) // K
    perm = jnp.argsort(flat_e, stable=True)
    tok_sorted = flat_tok[perm]
    counts = jnp.sum(flat_e[:, None] == jnp.arange(E)[None, :], axis=0)
    bounds = jnp.concatenate(
        [jnp.zeros((1,), jnp.int32), jnp.cumsum(counts).astype(jnp.int32)])
    pos = jnp.zeros((N,), jnp.int32).at[perm].set(
        jnp.arange(N, dtype=jnp.int32)).reshape(B, K)
    return tok_sorted, bounds, pos


def _work_units(bounds):
    cnt = bounds[1:] - bounds[:-1]
    t_lo = bounds[:-1] // TM
    t_hi = (bounds[1:] + TM - 1) // TM
    n_units = jnp.where(cnt > 0, t_hi - t_lo, 0)
    slot_end = jnp.cumsum(n_units)
    slot_start = slot_end - n_units
    total = slot_end[-1]
    u = jnp.arange(NW, dtype=jnp.int32)
    e_u = jnp.searchsorted(slot_end, u, side='right').astype(jnp.int32)
    valid = u < total
    e_u = jnp.clip(e_u, 0, E - 1)
    tile = t_lo[e_u] + (u - slot_start[e_u])
    tile = jnp.where(valid, tile, NT - 1).astype(jnp.int32)
    lo = jnp.clip(bounds[e_u] - tile * TM, 0, TM)
    hi = jnp.clip(bounds[e_u + 1] - tile * TM, 0, TM)
    lo = jnp.where(valid, lo, 0).astype(jnp.int32)
    hi = jnp.where(valid, hi, 0).astype(jnp.int32)
    prev_tile = jnp.concatenate([jnp.full((1,), -1, jnp.int32), tile[:-1]])
    first = (valid & (tile != prev_tile)).astype(jnp.int32)
    return tile, e_u.astype(jnp.int32), lo, hi, first


def kernel(x, Wb, bb, Wg, bg, Wn, bn, W1, b1, W2, b2):
    noise = jax.random.normal(jax.random.key(42), (B, E), dtype=jnp.float32)
    feats, idx, wts = _gate(x, Wb, bb, Wg, bg, Wn, bn, noise)
    tok_flat = jnp.arange(N, dtype=jnp.int32) // K
    tok_sorted, pos_flat, counts = _route_sc(idx.reshape(-1), tok_flat)
    pos = pos_flat.reshape(B, K)
    bounds = jnp.concatenate(
        [jnp.zeros((1,), jnp.int32), jnp.cumsum(counts).astype(jnp.int32)])
    tile_id, exp_id, lo, hi, first = _work_units(bounds)
    xs = feats[tok_sorted]
    ys = _ffn(xs, W1, b1, W2, b2, tile_id, exp_id, lo, hi, first)
    out = wts[:, 0:1] * ys[pos[:, 0]] + wts[:, 1:2] * ys[pos[:, 1]]
    return out


# speedup vs baseline: 1.9943x; 1.0963x over previous
import functools

import jax
import jax.numpy as jnp
from jax import lax
from jax.experimental import pallas as pl
from jax.experimental.pallas import tpu as pltpu
from jax.experimental.pallas import tpu_sc as plsc

E = 16
K = 2
TEMP = 1.5
MIN_NOISE = 0.01

B = 2048
D = 768
H = 3072
N = B * K
TM = 256
NT = N // TM
NW = NT + E - 1
TB = 256

NEG = -3.0e38


def _gate_body(x_ref, wb_ref, bb_ref, wg_ref, bg_ref, wn_ref, bn_ref,
               noise_ref, feats_ref, idx_ref, wts_ref):
    x = x_ref[...]
    feats = jnp.dot(x, wb_ref[...], preferred_element_type=jnp.float32)
    feats = feats + bb_ref[...]
    feats_ref[...] = feats
    raw = jnp.dot(feats, wg_ref[...], preferred_element_type=jnp.float32)
    raw = raw + bg_ref[...]
    nl = jnp.dot(feats, wn_ref[...], preferred_element_type=jnp.float32)
    nl = nl + bn_ref[...]
    sigma = jax.nn.softplus(nl) + MIN_NOISE
    scores = raw + sigma * noise_ref[...]
    iota = jax.lax.broadcasted_iota(jnp.int32, scores.shape, 1)
    m1 = jnp.max(scores, axis=1, keepdims=True)
    i1 = jnp.min(jnp.where(scores == m1, iota, E), axis=1, keepdims=True)
    s2 = jnp.where(iota == i1, NEG, scores)
    m2 = jnp.max(s2, axis=1, keepdims=True)
    i2 = jnp.min(jnp.where(s2 == m2, iota, E), axis=1, keepdims=True)
    e2 = jnp.exp((m2 - m1) / TEMP)
    w1 = 1.0 / (1.0 + e2)
    w2 = 1.0 - w1
    idx_ref[...] = jnp.concatenate([i1, i2], axis=1)
    wts_ref[...] = jnp.concatenate([w1, w2], axis=1)


def _gate(x, Wb, bb, Wg, bg, Wn, bn, noise):
    grid = (B // TB,)
    return pl.pallas_call(
        _gate_body,
        grid=grid,
        in_specs=[
            pl.BlockSpec((TB, D), lambda i: (i, 0)),
            pl.BlockSpec((D, D), lambda i: (0, 0)),
            pl.BlockSpec((1, D), lambda i: (0, 0)),
            pl.BlockSpec((D, E), lambda i: (0, 0)),
            pl.BlockSpec((1, E), lambda i: (0, 0)),
            pl.BlockSpec((D, E), lambda i: (0, 0)),
            pl.BlockSpec((1, E), lambda i: (0, 0)),
            pl.BlockSpec((TB, E), lambda i: (i, 0)),
        ],
        out_specs=[
            pl.BlockSpec((TB, D), lambda i: (i, 0)),
            pl.BlockSpec((TB, K), lambda i: (i, 0)),
            pl.BlockSpec((TB, K), lambda i: (i, 0)),
        ],
        out_shape=[
            jax.ShapeDtypeStruct((B, D), jnp.float32),
            jax.ShapeDtypeStruct((B, K), jnp.int32),
            jax.ShapeDtypeStruct((B, K), jnp.float32),
        ],
    )(x, Wb, bb.reshape(1, D), Wg, bg.reshape(1, E), Wn, bn.reshape(1, E),
      noise)


def _ffn_body(tile_ref, exp_ref, lo_ref, hi_ref, first_ref,
              xs_ref, w1_ref, b1_ref, w2_ref, b2_ref, ys_ref):
    u = pl.program_id(0)
    lo = lo_ref[u]
    hi = hi_ref[u]
    first = first_ref[u]
    xs = xs_ref[...]
    h = jnp.dot(xs, w1_ref[0], preferred_element_type=jnp.float32)
    h = jnp.maximum(h + b1_ref[0], 0.0)
    y = jnp.dot(h, w2_ref[0], preferred_element_type=jnp.float32)
    y = y + b2_ref[0]
    rows = jax.lax.broadcasted_iota(jnp.int32, (TM, D), 0)
    mask = (rows >= lo) & (rows < hi)
    prev = jnp.where(first == 1, 0.0, ys_ref[...])
    ys_ref[...] = jnp.where(mask, y, prev)


def _ffn(xs, W1, b1, W2, b2, tile_id, exp_id, lo, hi, first):
    grid_spec = pltpu.PrefetchScalarGridSpec(
        num_scalar_prefetch=5,
        grid=(NW,),
        in_specs=[
            pl.BlockSpec((TM, D), lambda u, t, e, l, h, f: (t[u], 0)),
            pl.BlockSpec((1, D, H), lambda u, t, e, l, h, f: (e[u], 0, 0)),
            pl.BlockSpec((1, 1, H), lambda u, t, e, l, h, f: (e[u], 0, 0)),
            pl.BlockSpec((1, H, D), lambda u, t, e, l, h, f: (e[u], 0, 0)),
            pl.BlockSpec((1, 1, D), lambda u, t, e, l, h, f: (e[u], 0, 0)),
        ],
        out_specs=pl.BlockSpec((TM, D), lambda u, t, e, l, h, f: (t[u], 0)),
    )
    return pl.pallas_call(
        _ffn_body,
        grid_spec=grid_spec,
        out_shape=jax.ShapeDtypeStruct((N, D), jnp.float32),
    )(tile_id, exp_id, lo, hi, first, xs, W1,
      b1.reshape(E, 1, H), W2, b2.reshape(E, 1, D))


NC = 2
NS = 16
CH = N // NS
CHR = CH // 128


def _route_sc_body(idx_hbm, tok_hbm, toks_hbm, pos_hbm, cnt_hbm,
                   chunk_v, dst_v, rank_v, hist_v, cnt_v,
                   tmp_v, tmp1d_v, sh_hist, sh_tok):
    c = lax.axis_index("c")
    s = lax.axis_index("s")
    base = s * CH

    @pl.when(c == 0)
    def _():
        iota = lax.iota(jnp.int32, NS)
        zc = jnp.zeros((16,), jnp.int32)
        ones = zc + 1

        def eq01(a, b):
            d = a - b
            return ones - jnp.minimum(d * d, ones)

        pltpu.sync_copy(idx_hbm.at[pl.ds(s * CHR, CHR)], chunk_v)
        hist = zc
        for k in range(CHR):
            for g in range(8):
                ev = chunk_v[k, pl.ds(g * 16, 16)]
                rank = zc
                for j in range(16):
                    bj = jnp.broadcast_to(ev[j], (16,))
                    gtj = jnp.minimum(jnp.maximum(iota - j, zc), ones)
                    hist = hist + eq01(iota, bj)
                    rank = rank + eq01(ev, bj) * gtj
                rank_v[k, pl.ds(g * 16, 16)] = rank
        hist_v[...] = hist
        pltpu.sync_copy(hist_v, sh_hist.at[s])
        plsc.subcore_barrier()
        sv = jnp.broadcast_to(s, (16,))
        basev = zc
        totv = zc
        for r in range(NS):
            pltpu.sync_copy(sh_hist.at[r], hist_v)
            row = hist_v[...]
            mine = jnp.minimum(jnp.maximum(sv - r, zc), ones)
            basev = basev + row * mine
            totv = totv + row
        excl = zc
        acc = totv[0] * 0
        for e in range(E):
            excl = excl + eq01(iota, zc + e) * jnp.broadcast_to(acc, (16,))
            acc = acc + totv[e]
        startv = excl + basev

        @pl.when(s == 0)
        def _():
            cnt_v[...] = totv
            pltpu.sync_copy(cnt_v, cnt_hbm)

        for k in range(CHR):
            for g in range(8):
                ev = chunk_v[k, pl.ds(g * 16, 16)]
                rank = rank_v[k, pl.ds(g * 16, 16)]
                startlane = zc
                histg = zc
                for e in range(E):
                    me = eq01(ev, zc + e)
                    startlane = startlane + me * jnp.broadcast_to(
                        startv[e], (16,))
                for j in range(16):
                    bj = jnp.broadcast_to(ev[j], (16,))
                    histg = histg + eq01(iota, bj)
                startv = startv + histg
                dst_v[k, pl.ds(g * 16, 16)] = startlane + rank
        pltpu.sync_copy(tok_hbm.at[pl.ds(s * CHR, CHR)], tmp_v)
        for k in range(CHR):
            pltpu.sync_copy(dst_v.at[k], pos_hbm.at[pl.ds(base + k * 128, 128)])
            pltpu.sync_copy(tmp_v.at[k], sh_tok.at[dst_v.at[k]])
        plsc.subcore_barrier()
        pltpu.sync_copy(sh_tok.at[pl.ds(base, CH)], tmp1d_v)
        pltpu.sync_copy(tmp1d_v, toks_hbm.at[pl.ds(base, CH)])


def _route_sc(idx_flat, tok_flat):
    mesh = plsc.VectorSubcoreMesh(core_axis_name="c", subcore_axis_name="s",
                                  num_cores=NC, num_subcores=NS)
    f = pl.kernel(
        _route_sc_body,
        out_type=[
            jax.ShapeDtypeStruct((N,), jnp.int32),
            jax.ShapeDtypeStruct((N,), jnp.int32),
            jax.ShapeDtypeStruct((E,), jnp.int32),
        ],
        mesh=mesh,
        scratch_types=[
            pltpu.VMEM((CHR, 128), jnp.int32),
            pltpu.VMEM((CHR, 128), jnp.int32),
            pltpu.VMEM((CHR, 128), jnp.int32),
            pltpu.VMEM((NS,), jnp.int32),
            pltpu.VMEM((E,), jnp.int32),
            pltpu.VMEM((CHR, 128), jnp.int32),
            pltpu.VMEM((CH,), jnp.int32),
            pltpu.VMEM_SHARED((NS, NS), jnp.int32),
            pltpu.VMEM_SHARED((N,), jnp.int32),
        ],
    )
    return f(idx_flat.reshape(N // 128, 128), tok_flat.reshape(N // 128, 128))


GW = NC * NS
GR = N // GW


def _gather_sc_body(feats_hbm, toks_hbm, xs_hbm, idx_v, rows_v, sem):
    wid = lax.axis_index("s") * NC + lax.axis_index("c")
    base = wid * GR
    pltpu.sync_copy(toks_hbm.at[pl.ds(base, GR)], idx_v)
    pltpu.async_copy(feats_hbm.at[idx_v], rows_v, sem).wait()
    pltpu.sync_copy(rows_v, xs_hbm.at[pl.ds(base, GR)])


def _gather_sc(feats, toks):
    mesh = plsc.VectorSubcoreMesh(core_axis_name="c", subcore_axis_name="s",
                                  num_cores=NC, num_subcores=NS)
    f = pl.kernel(
        _gather_sc_body,
        out_type=jax.ShapeDtypeStruct((N, D), jnp.float32),
        mesh=mesh,
        scratch_types=[
            pltpu.VMEM((GR,), jnp.int32),
            pltpu.VMEM((GR, D), jnp.float32),
            pltpu.SemaphoreType.DMA,
        ],
    )
    return f(feats, toks)


TPW = B // GW
TPH = TPW // 2


def _combine_sc_body(ys_hbm, pos_hbm, wts_hbm, out_hbm,
                     rows0_v, rows1_v, out_v, pidx0_v, pidx1_v, wts_v,
                     sem0, sem1):
    wid = lax.axis_index("s") * NC + lax.axis_index("c")
    tb = wid * TPW
    pltpu.sync_copy(wts_hbm.at[pl.ds(tb * K, TPW * K)],
                    wts_v.at[pl.ds(0, TPW * K)])
    pltpu.sync_copy(pos_hbm.at[pl.ds(tb * K, TPH * K)], pidx0_v)
    pltpu.sync_copy(pos_hbm.at[pl.ds(tb * K + TPH * K, TPH * K)], pidx1_v)
    cp0 = pltpu.async_copy(ys_hbm.at[pidx0_v], rows0_v, sem0)
    cp1 = pltpu.async_copy(ys_hbm.at[pidx1_v], rows1_v, sem1)
    cp0.wait()

    def _do_half(rows_v, half):
        def body(t, _):
            woff = half * TPH * K + t * K
            wpair = wts_v[pl.ds(woff, 16)]
            w0 = jnp.broadcast_to(wpair[0], (16,))
            w1 = jnp.broadcast_to(wpair[1], (16,))
            for v in range(D // 16):
                r0 = rows_v[t * K, pl.ds(v * 16, 16)]
                r1 = rows_v[t * K + 1, pl.ds(v * 16, 16)]
                out_v[t, pl.ds(v * 16, 16)] = r0 * w0 + r1 * w1
            return 0

        lax.fori_loop(0, TPH, body, 0)
        pltpu.sync_copy(out_v, out_hbm.at[pl.ds(tb + half * TPH, TPH)])

    _do_half(rows0_v, 0)
    cp1.wait()
    _do_half(rows1_v, 1)


def _combine_sc(ys, pos_flat, wts_flat):
    mesh = plsc.VectorSubcoreMesh(core_axis_name="c", subcore_axis_name="s",
                                  num_cores=NC, num_subcores=NS)
    f = pl.kernel(
        _combine_sc_body,
        out_type=jax.ShapeDtypeStruct((B, D), jnp.float32),
        mesh=mesh,
        scratch_types=[
            pltpu.VMEM((TPH * K, D), jnp.float32),
            pltpu.VMEM((TPH * K, D), jnp.float32),
            pltpu.VMEM((TPH, D), jnp.float32),
            pltpu.VMEM((TPH * K,), jnp.int32),
            pltpu.VMEM((TPH * K,), jnp.int32),
            pltpu.VMEM((TPW * K + 16,), jnp.float32),
            pltpu.SemaphoreType.DMA,
            pltpu.SemaphoreType.DMA,
        ],
    )
    return f(ys, pos_flat, wts_flat)


def _route(idx, wts):
    flat_e = idx.reshape(-1)
    flat_tok = jnp.arange(N, dtype=jnp.int32) // K
    perm = jnp.argsort(flat_e, stable=True)
    tok_sorted = flat_tok[perm]
    counts = jnp.sum(flat_e[:, None] == jnp.arange(E)[None, :], axis=0)
    bounds = jnp.concatenate(
        [jnp.zeros((1,), jnp.int32), jnp.cumsum(counts).astype(jnp.int32)])
    pos = jnp.zeros((N,), jnp.int32).at[perm].set(
        jnp.arange(N, dtype=jnp.int32)).reshape(B, K)
    return tok_sorted, bounds, pos


def _work_units(bounds):
    cnt = bounds[1:] - bounds[:-1]
    t_lo = bounds[:-1] // TM
    t_hi = (bounds[1:] + TM - 1) // TM
    n_units = jnp.where(cnt > 0, t_hi - t_lo, 0)
    slot_end = jnp.cumsum(n_units)
    slot_start = slot_end - n_units
    total = slot_end[-1]
    u = jnp.arange(NW, dtype=jnp.int32)
    e_u = jnp.searchsorted(slot_end, u, side='right').astype(jnp.int32)
    valid = u < total
    e_u = jnp.clip(e_u, 0, E - 1)
    tile = t_lo[e_u] + (u - slot_start[e_u])
    tile = jnp.where(valid, tile, NT - 1).astype(jnp.int32)
    lo = jnp.clip(bounds[e_u] - tile * TM, 0, TM)
    hi = jnp.clip(bounds[e_u + 1] - tile * TM, 0, TM)
    lo = jnp.where(valid, lo, 0).astype(jnp.int32)
    hi = jnp.where(valid, hi, 0).astype(jnp.int32)
    prev_tile = jnp.concatenate([jnp.full((1,), -1, jnp.int32), tile[:-1]])
    first = (valid & (tile != prev_tile)).astype(jnp.int32)
    return tile, e_u.astype(jnp.int32), lo, hi, first


_NOISE_NP = None


def _noise_const():
    global _NOISE_NP
    if _NOISE_NP is None:
        import numpy as _np
        with jax.ensure_compile_time_eval():
            _NOISE_NP = _np.asarray(
                jax.random.normal(jax.random.key(42), (B, E),
                                  dtype=jnp.float32))
    return jnp.asarray(_NOISE_NP)


def kernel(x, Wb, bb, Wg, bg, Wn, bn, W1, b1, W2, b2):
    noise = _noise_const()
    feats, idx, wts = _gate(x, Wb, bb, Wg, bg, Wn, bn, noise)
    tok_flat = jnp.arange(N, dtype=jnp.int32) // K
    tok_sorted, pos_flat, counts = _route_sc(idx.reshape(-1), tok_flat)
    bounds = jnp.concatenate(
        [jnp.zeros((1,), jnp.int32), jnp.cumsum(counts).astype(jnp.int32)])
    tile_id, exp_id, lo, hi, first = _work_units(bounds)
    xs = _gather_sc(feats, tok_sorted)
    ys = _ffn(xs, W1, b1, W2, b2, tile_id, exp_id, lo, hi, first)
    return _combine_sc(ys, pos_flat, wts.reshape(-1))


# speedup vs baseline: 2.0054x; 1.0056x over previous
import functools

import jax
import jax.numpy as jnp
from jax import lax
from jax.experimental import pallas as pl
from jax.experimental.pallas import tpu as pltpu
from jax.experimental.pallas import tpu_sc as plsc

E = 16
K = 2
TEMP = 1.5
MIN_NOISE = 0.01

B = 2048
D = 768
H = 3072
N = B * K
TM = 256
NT = N // TM
NW = NT + E - 1
TB = 256

NEG = -3.0e38


def _gate_body(x_ref, wb_ref, bb_ref, wg_ref, bg_ref, wn_ref, bn_ref,
               noise_ref, feats_ref, idx_ref, wts_ref):
    x = x_ref[...]
    feats = jnp.dot(x, wb_ref[...], preferred_element_type=jnp.float32)
    feats = feats + bb_ref[...]
    feats_ref[...] = feats
    raw = jnp.dot(feats, wg_ref[...], preferred_element_type=jnp.float32)
    raw = raw + bg_ref[...]
    nl = jnp.dot(feats, wn_ref[...], preferred_element_type=jnp.float32)
    nl = nl + bn_ref[...]
    sigma = jax.nn.softplus(nl) + MIN_NOISE
    scores = raw + sigma * noise_ref[...]
    iota = jax.lax.broadcasted_iota(jnp.int32, scores.shape, 1)
    m1 = jnp.max(scores, axis=1, keepdims=True)
    i1 = jnp.min(jnp.where(scores == m1, iota, E), axis=1, keepdims=True)
    s2 = jnp.where(iota == i1, NEG, scores)
    m2 = jnp.max(s2, axis=1, keepdims=True)
    i2 = jnp.min(jnp.where(s2 == m2, iota, E), axis=1, keepdims=True)
    e2 = jnp.exp((m2 - m1) / TEMP)
    w1 = 1.0 / (1.0 + e2)
    w2 = 1.0 - w1
    idx_ref[...] = jnp.concatenate([i1, i2], axis=1)
    wts_ref[...] = jnp.concatenate([w1, w2], axis=1)


def _gate(x, Wb, bb, Wg, bg, Wn, bn, noise):
    grid = (B // TB,)
    return pl.pallas_call(
        _gate_body,
        grid=grid,
        in_specs=[
            pl.BlockSpec((TB, D), lambda i: (i, 0)),
            pl.BlockSpec((D, D), lambda i: (0, 0)),
            pl.BlockSpec((1, D), lambda i: (0, 0)),
            pl.BlockSpec((D, E), lambda i: (0, 0)),
            pl.BlockSpec((1, E), lambda i: (0, 0)),
            pl.BlockSpec((D, E), lambda i: (0, 0)),
            pl.BlockSpec((1, E), lambda i: (0, 0)),
            pl.BlockSpec((TB, E), lambda i: (i, 0)),
        ],
        out_specs=[
            pl.BlockSpec((TB, D), lambda i: (i, 0)),
            pl.BlockSpec((TB, K), lambda i: (i, 0)),
            pl.BlockSpec((TB, K), lambda i: (i, 0)),
        ],
        out_shape=[
            jax.ShapeDtypeStruct((B, D), jnp.float32),
            jax.ShapeDtypeStruct((B, K), jnp.int32),
            jax.ShapeDtypeStruct((B, K), jnp.float32),
        ],
    )(x, Wb, bb.reshape(1, D), Wg, bg.reshape(1, E), Wn, bn.reshape(1, E),
      noise)


def _ffn_body(tile_ref, exp_ref, lo_ref, hi_ref, first_ref,
              xs_ref, w1_ref, b1_ref, w2_ref, b2_ref, wc_ref, ys_ref):
    u = pl.program_id(0)
    lo = lo_ref[u]
    hi = hi_ref[u]
    first = first_ref[u]
    xs = xs_ref[...]
    h = jnp.dot(xs, w1_ref[0], preferred_element_type=jnp.float32)
    h = jnp.maximum(h + b1_ref[0], 0.0)
    y = jnp.dot(h, w2_ref[0], preferred_element_type=jnp.float32)
    y = (y + b2_ref[0]) * wc_ref[...]
    rows = jax.lax.broadcasted_iota(jnp.int32, (TM, D), 0)
    mask = (rows >= lo) & (rows < hi)
    prev = jnp.where(first == 1, 0.0, ys_ref[...])
    ys_ref[...] = jnp.where(mask, y, prev)


def _ffn(xs, W1, b1, W2, b2, ws, tile_id, exp_id, lo, hi, first):
    grid_spec = pltpu.PrefetchScalarGridSpec(
        num_scalar_prefetch=5,
        grid=(NW,),
        in_specs=[
            pl.BlockSpec((TM, D), lambda u, t, e, l, h, f: (t[u], 0)),
            pl.BlockSpec((1, D, H), lambda u, t, e, l, h, f: (e[u], 0, 0)),
            pl.BlockSpec((1, 1, H), lambda u, t, e, l, h, f: (e[u], 0, 0)),
            pl.BlockSpec((1, H, D), lambda u, t, e, l, h, f: (e[u], 0, 0)),
            pl.BlockSpec((1, 1, D), lambda u, t, e, l, h, f: (e[u], 0, 0)),
            pl.BlockSpec((TM, 1), lambda u, t, e, l, h, f: (t[u], 0)),
        ],
        out_specs=pl.BlockSpec((TM, D), lambda u, t, e, l, h, f: (t[u], 0)),
    )
    return pl.pallas_call(
        _ffn_body,
        grid_spec=grid_spec,
        out_shape=jax.ShapeDtypeStruct((N, D), jnp.float32),
    )(tile_id, exp_id, lo, hi, first, xs, W1,
      b1.reshape(E, 1, H), W2, b2.reshape(E, 1, D), ws.reshape(N, 1))


NC = 2
NS = 16
CH = N // NS
CHR = CH // 128


def _route_sc_body(idx_hbm, tok_hbm, wts_hbm, toks_hbm, ws_hbm, pos_hbm,
                   cnt_hbm, chunk_v, dst_v, rank_v, hist_v, cnt_v,
                   tmp_v, tmp1d_v, wtmp_v, wtmp1d_v, sh_hist, sh_tok, sh_w):
    c = lax.axis_index("c")
    s = lax.axis_index("s")
    base = s * CH

    @pl.when(c == 0)
    def _():
        iota = lax.iota(jnp.int32, NS)
        zc = jnp.zeros((16,), jnp.int32)
        ones = zc + 1

        def eq01(a, b):
            d = a - b
            return ones - jnp.minimum(d * d, ones)

        pltpu.sync_copy(idx_hbm.at[pl.ds(s * CHR, CHR)], chunk_v)
        hist = zc
        for k in range(CHR):
            for g in range(8):
                ev = chunk_v[k, pl.ds(g * 16, 16)]
                rank = zc
                for j in range(16):
                    bj = jnp.broadcast_to(ev[j], (16,))
                    gtj = jnp.minimum(jnp.maximum(iota - j, zc), ones)
                    hist = hist + eq01(iota, bj)
                    rank = rank + eq01(ev, bj) * gtj
                rank_v[k, pl.ds(g * 16, 16)] = rank
        hist_v[...] = hist
        pltpu.sync_copy(hist_v, sh_hist.at[s])
        plsc.subcore_barrier()
        sv = jnp.broadcast_to(s, (16,))
        basev = zc
        totv = zc
        for r in range(NS):
            pltpu.sync_copy(sh_hist.at[r], hist_v)
            row = hist_v[...]
            mine = jnp.minimum(jnp.maximum(sv - r, zc), ones)
            basev = basev + row * mine
            totv = totv + row
        excl = zc
        acc = totv[0] * 0
        for e in range(E):
            excl = excl + eq01(iota, zc + e) * jnp.broadcast_to(acc, (16,))
            acc = acc + totv[e]
        startv = excl + basev

        @pl.when(s == 0)
        def _():
            cnt_v[...] = totv
            pltpu.sync_copy(cnt_v, cnt_hbm)

        for k in range(CHR):
            for g in range(8):
                ev = chunk_v[k, pl.ds(g * 16, 16)]
                rank = rank_v[k, pl.ds(g * 16, 16)]
                startlane = zc
                histg = zc
                for e in range(E):
                    me = eq01(ev, zc + e)
                    startlane = startlane + me * jnp.broadcast_to(
                        startv[e], (16,))
                for j in range(16):
                    bj = jnp.broadcast_to(ev[j], (16,))
                    histg = histg + eq01(iota, bj)
                startv = startv + histg
                dst_v[k, pl.ds(g * 16, 16)] = startlane + rank
        pltpu.sync_copy(tok_hbm.at[pl.ds(s * CHR, CHR)], tmp_v)
        pltpu.sync_copy(wts_hbm.at[pl.ds(s * CHR, CHR)], wtmp_v)
        for k in range(CHR):
            pltpu.sync_copy(dst_v.at[k], pos_hbm.at[pl.ds(base + k * 128, 128)])
            pltpu.sync_copy(tmp_v.at[k], sh_tok.at[dst_v.at[k]])
            pltpu.sync_copy(wtmp_v.at[k], sh_w.at[dst_v.at[k]])
        plsc.subcore_barrier()
        pltpu.sync_copy(sh_tok.at[pl.ds(base, CH)], tmp1d_v)
        pltpu.sync_copy(tmp1d_v, toks_hbm.at[pl.ds(base, CH)])
        pltpu.sync_copy(sh_w.at[pl.ds(base, CH)], wtmp1d_v)
        pltpu.sync_copy(wtmp1d_v, ws_hbm.at[pl.ds(base, CH)])


def _route_sc(idx_flat, tok_flat, wts_flat):
    mesh = plsc.VectorSubcoreMesh(core_axis_name="c", subcore_axis_name="s",
                                  num_cores=NC, num_subcores=NS)
    f = pl.kernel(
        _route_sc_body,
        out_type=[
            jax.ShapeDtypeStruct((N,), jnp.int32),
            jax.ShapeDtypeStruct((N,), jnp.float32),
            jax.ShapeDtypeStruct((N,), jnp.int32),
            jax.ShapeDtypeStruct((E,), jnp.int32),
        ],
        mesh=mesh,
        scratch_types=[
            pltpu.VMEM((CHR, 128), jnp.int32),
            pltpu.VMEM((CHR, 128), jnp.int32),
            pltpu.VMEM((CHR, 128), jnp.int32),
            pltpu.VMEM((NS,), jnp.int32),
            pltpu.VMEM((E,), jnp.int32),
            pltpu.VMEM((CHR, 128), jnp.int32),
            pltpu.VMEM((CH,), jnp.int32),
            pltpu.VMEM((CHR, 128), jnp.float32),
            pltpu.VMEM((CH,), jnp.float32),
            pltpu.VMEM_SHARED((NS, NS), jnp.int32),
            pltpu.VMEM_SHARED((N,), jnp.int32),
            pltpu.VMEM_SHARED((N,), jnp.float32),
        ],
    )
    return f(idx_flat.reshape(N // 128, 128), tok_flat.reshape(N // 128, 128),
             wts_flat.reshape(N // 128, 128))


GW = NC * NS
GR = N // GW


def _gather_sc_body(feats_hbm, toks_hbm, xs_hbm, idx_v, rows_v, sem):
    wid = lax.axis_index("s") * NC + lax.axis_index("c")
    base = wid * GR
    pltpu.sync_copy(toks_hbm.at[pl.ds(base, GR)], idx_v)
    pltpu.async_copy(feats_hbm.at[idx_v], rows_v, sem).wait()
    pltpu.sync_copy(rows_v, xs_hbm.at[pl.ds(base, GR)])


def _gather_sc(feats, toks):
    mesh = plsc.VectorSubcoreMesh(core_axis_name="c", subcore_axis_name="s",
                                  num_cores=NC, num_subcores=NS)
    f = pl.kernel(
        _gather_sc_body,
        out_type=jax.ShapeDtypeStruct((N, D), jnp.float32),
        mesh=mesh,
        scratch_types=[
            pltpu.VMEM((GR,), jnp.int32),
            pltpu.VMEM((GR, D), jnp.float32),
            pltpu.SemaphoreType.DMA,
        ],
    )
    return f(feats, toks)


TPW = B // GW
TPH = TPW // 2


def _combine_sc_body(ys_hbm, pos_hbm, out_hbm,
                     rows0_v, rows1_v, out_v, pidx0_v, pidx1_v,
                     sem0, sem1):
    wid = lax.axis_index("s") * NC + lax.axis_index("c")
    tb = wid * TPW
    pltpu.sync_copy(pos_hbm.at[pl.ds(tb * K, TPH * K)], pidx0_v)
    pltpu.sync_copy(pos_hbm.at[pl.ds(tb * K + TPH * K, TPH * K)], pidx1_v)
    cp0 = pltpu.async_copy(ys_hbm.at[pidx0_v], rows0_v, sem0)
    cp1 = pltpu.async_copy(ys_hbm.at[pidx1_v], rows1_v, sem1)
    cp0.wait()

    def _do_half(rows_v, half):
        def body(t, _):
            for v in range(D // 16):
                r0 = rows_v[t * K, pl.ds(v * 16, 16)]
                r1 = rows_v[t * K + 1, pl.ds(v * 16, 16)]
                out_v[t, pl.ds(v * 16, 16)] = r0 + r1
            return 0

        lax.fori_loop(0, TPH, body, 0)
        pltpu.sync_copy(out_v, out_hbm.at[pl.ds(tb + half * TPH, TPH)])

    _do_half(rows0_v, 0)
    cp1.wait()
    _do_half(rows1_v, 1)


def _combine_sc(ys, pos_flat):
    mesh = plsc.VectorSubcoreMesh(core_axis_name="c", subcore_axis_name="s",
                                  num_cores=NC, num_subcores=NS)
    f = pl.kernel(
        _combine_sc_body,
        out_type=jax.ShapeDtypeStruct((B, D), jnp.float32),
        mesh=mesh,
        scratch_types=[
            pltpu.VMEM((TPH * K, D), jnp.float32),
            pltpu.VMEM((TPH * K, D), jnp.float32),
            pltpu.VMEM((TPH, D), jnp.float32),
            pltpu.VMEM((TPH * K,), jnp.int32),
            pltpu.VMEM((TPH * K,), jnp.int32),
            pltpu.SemaphoreType.DMA,
            pltpu.SemaphoreType.DMA,
        ],
    )
    return f(ys, pos_flat)


def _route(idx, wts):
    flat_e = idx.reshape(-1)
    flat_tok = jnp.arange(N, dtype=jnp.int32) // K
    perm = jnp.argsort(flat_e, stable=True)
    tok_sorted = flat_tok[perm]
    counts = jnp.sum(flat_e[:, None] == jnp.arange(E)[None, :], axis=0)
    bounds = jnp.concatenate(
        [jnp.zeros((1,), jnp.int32), jnp.cumsum(counts).astype(jnp.int32)])
    pos = jnp.zeros((N,), jnp.int32).at[perm].set(
        jnp.arange(N, dtype=jnp.int32)).reshape(B, K)
    return tok_sorted, bounds, pos


def _work_units(bounds):
    cnt = bounds[1:] - bounds[:-1]
    t_lo = bounds[:-1] // TM
    t_hi = (bounds[1:] + TM - 1) // TM
    n_units = jnp.where(cnt > 0, t_hi - t_lo, 0)
    slot_end = jnp.cumsum(n_units)
    slot_start = slot_end - n_units
    total = slot_end[-1]
    u = jnp.arange(NW, dtype=jnp.int32)
    e_u = jnp.searchsorted(slot_end, u, side='right').astype(jnp.int32)
    valid = u < total
    e_u = jnp.clip(e_u, 0, E - 1)
    tile = t_lo[e_u] + (u - slot_start[e_u])
    tile = jnp.where(valid, tile, NT - 1).astype(jnp.int32)
    lo = jnp.clip(bounds[e_u] - tile * TM, 0, TM)
    hi = jnp.clip(bounds[e_u + 1] - tile * TM, 0, TM)
    lo = jnp.where(valid, lo, 0).astype(jnp.int32)
    hi = jnp.where(valid, hi, 0).astype(jnp.int32)
    prev_tile = jnp.concatenate([jnp.full((1,), -1, jnp.int32), tile[:-1]])
    first = (valid & (tile != prev_tile)).astype(jnp.int32)
    return tile, e_u.astype(jnp.int32), lo, hi, first


_NOISE_NP = None


def _noise_const():
    global _NOISE_NP
    if _NOISE_NP is None:
        import numpy as _np
        cpu = jax.local_devices(backend="cpu")[0]
        with jax.ensure_compile_time_eval(), jax.default_device(cpu):
            _NOISE_NP = _np.asarray(
                jax.random.normal(jax.random.key(42), (B, E),
                                  dtype=jnp.float32))
    return jnp.asarray(_NOISE_NP)


def kernel(x, Wb, bb, Wg, bg, Wn, bn, W1, b1, W2, b2):
    import numpy as _np
    noise = _noise_const()
    tok_flat = jnp.asarray(_np.arange(N, dtype=_np.int32) // K)
    feats, idx, wts = _gate(x, Wb, bb, Wg, bg, Wn, bn, noise)
    tok_sorted, ws, pos_flat, counts = _route_sc(
        idx.reshape(-1), tok_flat, wts.reshape(-1))
    bounds = jnp.concatenate(
        [jnp.zeros((1,), jnp.int32), jnp.cumsum(counts).astype(jnp.int32)])
    tile_id, exp_id, lo, hi, first = _work_units(bounds)
    xs = _gather_sc(feats, tok_sorted)
    ys = _ffn(xs, W1, b1, W2, b2, ws, tile_id, exp_id, lo, hi, first)
    return _combine_sc(ys, pos_flat)
